# baseline (speedup 1.0000x reference)
import functools

import jax
import jax.numpy as jnp
from jax import lax
from jax.experimental import pallas as pl
from jax.experimental.pallas import tpu as pltpu
from jax.experimental.pallas import tpu_sc as plsc

N = 10000
E = 320000
D = 128
GAMMA = 1.0

NC = 2
NS = 16

PAD_N = 10240
HIST_SLICE = PAD_N // NS
ROWS_PER_TILE = PAD_N // NS
LAST_ROWS = N - (NS - 1) * ROWS_PER_TILE

E_PER_TILE = E // NS
K = 80
CHUNKS = E_PER_TILE // K
SEG = 50
SEG_E = SEG * K

_mesh = plsc.VectorSubcoreMesh(
    core_axis_name="c", subcore_axis_name="s", num_cores=NC, num_subcores=NS)


def _copy_idx_chunk(big, small, base):
  for i in range(K // 16):
    small[pl.ds(i * 16, 16)] = big[pl.ds(base + i * 16, 16)]


@functools.partial(
    pl.kernel,
    out_type=jax.ShapeDtypeStruct((2 * PAD_N,), jnp.float32),
    mesh=_mesh,
    scratch_types=[
        pltpu.VMEM((E_PER_TILE,), jnp.int32),
        pltpu.VMEM((K,), jnp.int32),
        pltpu.VMEM((K,), jnp.int32),
        pltpu.VMEM((K,), jnp.float32),
        pltpu.VMEM((HIST_SLICE,), jnp.float32),
        pltpu.VMEM_SHARED((PAD_N,), jnp.float32),
        pltpu.SemaphoreType.DMA,
        pltpu.SemaphoreType.DMA,
    ],
)
def _deg_kernel(edges_flat, deg_out, idx_big, idx0, idx1, ones_v, zeros_v,
                hist_sh, sem0, sem1):
  c = lax.axis_index("c")
  s = lax.axis_index("s")
  idx_small = (idx0, idx1)
  sems = (sem0, sem1)

  for i in range(K // 16):
    ones_v[pl.ds(i * 16, 16)] = jnp.full((16,), 1.0, jnp.float32)
  for i in range(HIST_SLICE // 16):
    zeros_v[pl.ds(i * 16, 16)] = jnp.full((16,), 0.0, jnp.float32)
  pltpu.sync_copy(zeros_v, hist_sh.at[pl.ds(s * HIST_SLICE, HIST_SLICE)])
  pltpu.sync_copy(edges_flat.at[pl.ds(c * E + s * E_PER_TILE, E_PER_TILE)],
                  idx_big)
  plsc.subcore_barrier()

  def fire(b):
    pltpu.async_copy(ones_v, hist_sh.at[idx_small[b]], sems[b], add=True)

  def drain(b):
    pltpu.make_async_copy(ones_v, hist_sh.at[idx_small[b]], sems[b]).wait()

  for b in range(2):
    _copy_idx_chunk(idx_big, idx_small[b], b * K)
    fire(b)

  def outer(o, carry):
    for b in range(2):
      k = o * 2 + b
      drain(b)
      kn = k + 2

      @pl.when(kn < CHUNKS)
      def _():
        _copy_idx_chunk(idx_big, idx_small[b], kn * K)
        fire(b)

    return carry

  lax.fori_loop(0, CHUNKS // 2, outer, 0)
  plsc.subcore_barrier()
  pltpu.sync_copy(hist_sh.at[pl.ds(s * HIST_SLICE, HIST_SLICE)],
                  deg_out.at[pl.ds(c * PAD_N + s * HIST_SLICE, HIST_SLICE)])


BLK = 2000
GRID_M = N // BLK


def _dense_body(feat_ref, wm_ref, wv_ref, deg_ref, gm_ref, gv_ref):
  x = feat_ref[...]
  mean = jnp.maximum(jnp.dot(x, wm_ref[...],
                             preferred_element_type=jnp.float32), 0.0)
  var = jnp.maximum(jnp.dot(x, wv_ref[...],
                            preferred_element_type=jnp.float32), 0.0)
  att = jnp.exp(-GAMMA * var)
  a = lax.rsqrt(deg_ref[...][:, 0:1] + 1.0)
  gm_ref[...] = a * (mean * att)
  gv_ref[...] = (a * a) * (var * att * att)


_dense_call = pl.pallas_call(
    _dense_body,
    grid=(GRID_M,),
    in_specs=[
        pl.BlockSpec((BLK, D), lambda i: (i, 0)),
        pl.BlockSpec((D, D), lambda i: (0, 0)),
        pl.BlockSpec((D, D), lambda i: (0, 0)),
        pl.BlockSpec((BLK, 2), lambda i: (i, 0)),
    ],
    out_specs=[
        pl.BlockSpec((BLK, D), lambda i: (i, 0)),
        pl.BlockSpec((BLK, D), lambda i: (i, 0)),
    ],
    out_shape=[
        jax.ShapeDtypeStruct((N, D), jnp.float32),
        jax.ShapeDtypeStruct((N, D), jnp.float32),
    ],
)


@functools.partial(
    pl.kernel,
    out_type=jax.ShapeDtypeStruct((2, PAD_N, D), jnp.float32),
    mesh=_mesh,
    scratch_types=[
        pltpu.VMEM((SEG_E,), jnp.int32),
        pltpu.VMEM((SEG_E,), jnp.int32),
        pltpu.VMEM((K,), jnp.int32),
        pltpu.VMEM((K,), jnp.int32),
        pltpu.VMEM((K,), jnp.int32),
        pltpu.VMEM((K,), jnp.int32),
        pltpu.VMEM((K,), jnp.int32),
        pltpu.VMEM((K,), jnp.int32),
        pltpu.VMEM((K,), jnp.int32),
        pltpu.VMEM((K,), jnp.int32),
        pltpu.VMEM((K, D), jnp.float32),
        pltpu.VMEM((K, D), jnp.float32),
        pltpu.VMEM((K, D), jnp.float32),
        pltpu.VMEM((K, D), jnp.float32),
        pltpu.VMEM_SHARED((N, D), jnp.float32),
        pltpu.SemaphoreType.DMA,
        pltpu.SemaphoreType.DMA,
        pltpu.SemaphoreType.DMA,
        pltpu.SemaphoreType.DMA,
        pltpu.SemaphoreType.DMA,
        pltpu.SemaphoreType.DMA,
        pltpu.SemaphoreType.DMA,
        pltpu.SemaphoreType.DMA,
    ],
)
def _agg_kernel(src_hbm, dst_hbm, gm, gv, acc_out, src_seg, dst_seg,
                sidx0, sidx1, sidx2, sidx3, didx0, didx1, didx2, didx3,
                rows0, rows1, rows2, rows3,
                acc_sh, gsem0, gsem1, gsem2, gsem3,
                ssem0, ssem1, ssem2, ssem3):
  c = lax.axis_index("c")
  s = lax.axis_index("s")
  sidx = (sidx0, sidx1, sidx2, sidx3)
  didx = (didx0, didx1, didx2, didx3)
  rows = (rows0, rows1, rows2, rows3)
  gsems = (gsem0, gsem1, gsem2, gsem3)
  ssems = (ssem0, ssem1, ssem2, ssem3)

  rsl = pl.ds(s * ROWS_PER_TILE, ROWS_PER_TILE)
  lsl = pl.ds((NS - 1) * ROWS_PER_TILE, LAST_ROWS)

  @pl.when(jnp.logical_and(c == 0, s < NS - 1))
  def _():
    pltpu.sync_copy(gm.at[rsl], acc_sh.at[rsl])

  @pl.when(jnp.logical_and(c == 0, s == NS - 1))
  def _():
    pltpu.sync_copy(gm.at[lsl], acc_sh.at[lsl])

  @pl.when(jnp.logical_and(c == 1, s < NS - 1))
  def _():
    pltpu.sync_copy(gv.at[rsl], acc_sh.at[rsl])

  @pl.when(jnp.logical_and(c == 1, s == NS - 1))
  def _():
    pltpu.sync_copy(gv.at[lsl], acc_sh.at[lsl])

  plsc.subcore_barrier()

  def refill(seg_i):
    ssl = pl.ds(s * E_PER_TILE + seg_i * SEG_E, SEG_E)
    pltpu.sync_copy(src_hbm.at[ssl], src_seg)
    pltpu.sync_copy(dst_hbm.at[ssl], dst_seg)

  def stage(b, k):
    base = (k % SEG) * K
    _copy_idx_chunk(src_seg, sidx[b], base)
    _copy_idx_chunk(dst_seg, didx[b], base)

  def gather_start(b):
    @pl.when(c == 0)
    def _():
      pltpu.async_copy(gm.at[sidx[b]], rows[b], gsems[b])

    @pl.when(c == 1)
    def _():
      pltpu.async_copy(gv.at[sidx[b]], rows[b], gsems[b])

  def gather_wait(b):
    pltpu.make_async_copy(gm.at[sidx[b]], rows[b], gsems[b]).wait()

  def scatter_start(b):
    pltpu.async_copy(rows[b], acc_sh.at[didx[b]], ssems[b], add=True)

  def scatter_wait(b):
    pltpu.make_async_copy(rows[b], acc_sh.at[didx[b]], ssems[b]).wait()

  refill(0)
  for b in range(2):
    stage(b, b)
    gather_start(b)

  def chunk_body(k, q):
    qn = (q + 2) % 4
    gather_wait(q)

    @pl.when(k >= 2)
    def _():
      scatter_wait(qn)

    scatter_start(q)
    kn = k + 2

    @pl.when(kn < CHUNKS)
    def _():
      @pl.when(kn % SEG == 0)
      def _():
        refill(kn // SEG)

      stage(qn, kn)
      gather_start(qn)

  def outer(o, carry):
    for b in range(4):
      chunk_body(o * 4 + b, b)
    return carry

  lax.fori_loop(0, CHUNKS // 4, outer, 0)
  chunk_body(CHUNKS - 2, 0)
  chunk_body(CHUNKS - 1, 1)
  scatter_wait(0)
  scatter_wait(1)
  plsc.subcore_barrier()

  @pl.when(s < NS - 1)
  def _():
    pltpu.sync_copy(acc_sh.at[rsl], acc_out.at[c, rsl])

  @pl.when(s == NS - 1)
  def _():
    pltpu.sync_copy(acc_sh.at[lsl], acc_out.at[c, lsl])


def _final_body(acc_ref, deg_ref, om_ref, ov_ref):
  b = lax.rsqrt(deg_ref[...][:, 1:2] + 1.0)
  om_ref[...] = b * acc_ref[0]
  ov_ref[...] = (b * b) * acc_ref[1]


_final_call = pl.pallas_call(
    _final_body,
    grid=(GRID_M,),
    in_specs=[
        pl.BlockSpec((2, BLK, D), lambda i: (0, i, 0)),
        pl.BlockSpec((BLK, 2), lambda i: (i, 0)),
    ],
    out_specs=[
        pl.BlockSpec((BLK, D), lambda i: (i, 0)),
        pl.BlockSpec((BLK, D), lambda i: (i, 0)),
    ],
    out_shape=[
        jax.ShapeDtypeStruct((N, D), jnp.float32),
        jax.ShapeDtypeStruct((N, D), jnp.float32),
    ],
)


def kernel(feat, edge_index, weight_mean, weight_var):
  src = edge_index[0]
  dst = edge_index[1]
  deg = _deg_kernel(edge_index.reshape(-1))
  degcat = jnp.stack([deg[:N], deg[PAD_N:PAD_N + N]], axis=1)
  g_mean, g_var = _dense_call(feat, weight_mean, weight_var, degcat)
  acc = _agg_kernel(src, dst, g_mean, g_var)
  out_mean, out_var = _final_call(acc, degcat)
  return (out_mean, out_var)


# speedup vs baseline: 1.1165x; 1.1165x over previous
import functools

import jax
import jax.numpy as jnp
from jax import lax
from jax.experimental import pallas as pl
from jax.experimental.pallas import tpu as pltpu
from jax.experimental.pallas import tpu_sc as plsc

N = 10000
E = 320000
D = 128
GAMMA = 1.0

NC = 2
NS = 16

PAD_N = 10240
HIST_SLICE = PAD_N // NS
ROWS_PER_TILE = PAD_N // NS
LAST_ROWS = N - (NS - 1) * ROWS_PER_TILE

E_PER_TILE = E // NS
K = 80
CHUNKS = E_PER_TILE // K
SEG = 50
SEG_E = SEG * K

_mesh = plsc.VectorSubcoreMesh(
    core_axis_name="c", subcore_axis_name="s", num_cores=NC, num_subcores=NS)


def _copy_idx_chunk(big, small, base):
  for i in range(K // 16):
    small[pl.ds(i * 16, 16)] = big[pl.ds(base + i * 16, 16)]


@functools.partial(
    pl.kernel,
    out_type=jax.ShapeDtypeStruct((2 * PAD_N,), jnp.float32),
    mesh=_mesh,
    scratch_types=[
        pltpu.VMEM((E_PER_TILE,), jnp.int32),
        pltpu.VMEM((K,), jnp.int32),
        pltpu.VMEM((K,), jnp.int32),
        pltpu.VMEM((K,), jnp.float32),
        pltpu.VMEM((HIST_SLICE,), jnp.float32),
        pltpu.VMEM_SHARED((PAD_N,), jnp.float32),
        pltpu.SemaphoreType.DMA,
        pltpu.SemaphoreType.DMA,
    ],
)
def _deg_kernel(edges_flat, deg_out, idx_big, idx0, idx1, ones_v, zeros_v,
                hist_sh, sem0, sem1):
  c = lax.axis_index("c")
  s = lax.axis_index("s")
  idx_small = (idx0, idx1)
  sems = (sem0, sem1)

  for i in range(K // 16):
    ones_v[pl.ds(i * 16, 16)] = jnp.full((16,), 1.0, jnp.float32)
  for i in range(HIST_SLICE // 16):
    zeros_v[pl.ds(i * 16, 16)] = jnp.full((16,), 0.0, jnp.float32)
  pltpu.sync_copy(zeros_v, hist_sh.at[pl.ds(s * HIST_SLICE, HIST_SLICE)])
  pltpu.sync_copy(edges_flat.at[pl.ds(c * E + s * E_PER_TILE, E_PER_TILE)],
                  idx_big)
  plsc.subcore_barrier()

  def fire(b):
    pltpu.async_copy(ones_v, hist_sh.at[idx_small[b]], sems[b], add=True)

  def drain(b):
    pltpu.make_async_copy(ones_v, hist_sh.at[idx_small[b]], sems[b]).wait()

  for b in range(2):
    _copy_idx_chunk(idx_big, idx_small[b], b * K)
    fire(b)

  def outer(o, carry):
    for b in range(2):
      k = o * 2 + b
      drain(b)
      kn = k + 2

      @pl.when(kn < CHUNKS)
      def _():
        _copy_idx_chunk(idx_big, idx_small[b], kn * K)
        fire(b)

    return carry

  lax.fori_loop(0, CHUNKS // 2, outer, 0)
  plsc.subcore_barrier()
  pltpu.sync_copy(hist_sh.at[pl.ds(s * HIST_SLICE, HIST_SLICE)],
                  deg_out.at[pl.ds(c * PAD_N + s * HIST_SLICE, HIST_SLICE)])


BLK = 2000
GRID_M = N // BLK


def _dense_body(feat_ref, wm_ref, wv_ref, deg_ref, gm_ref, gv_ref):
  x = feat_ref[...]
  mean = jnp.maximum(jnp.dot(x, wm_ref[...],
                             preferred_element_type=jnp.float32), 0.0)
  var = jnp.maximum(jnp.dot(x, wv_ref[...],
                            preferred_element_type=jnp.float32), 0.0)
  att = jnp.exp(-GAMMA * var)
  a = lax.rsqrt(deg_ref[...][:, 0:1] + 1.0)
  gm_ref[...] = a * (mean * att)
  gv_ref[...] = (a * a) * (var * att * att)


_dense_call = pl.pallas_call(
    _dense_body,
    grid=(GRID_M,),
    in_specs=[
        pl.BlockSpec((BLK, D), lambda i: (i, 0)),
        pl.BlockSpec((D, D), lambda i: (0, 0)),
        pl.BlockSpec((D, D), lambda i: (0, 0)),
        pl.BlockSpec((BLK, 2), lambda i: (i, 0)),
    ],
    out_specs=[
        pl.BlockSpec((BLK, D), lambda i: (i, 0)),
        pl.BlockSpec((BLK, D), lambda i: (i, 0)),
    ],
    out_shape=[
        jax.ShapeDtypeStruct((N, D), jnp.float32),
        jax.ShapeDtypeStruct((N, D), jnp.float32),
    ],
)


@functools.partial(
    pl.kernel,
    out_type=jax.ShapeDtypeStruct((2, PAD_N, D), jnp.float32),
    mesh=_mesh,
    scratch_types=[
        pltpu.VMEM((SEG_E,), jnp.int32),
        pltpu.VMEM((SEG_E,), jnp.int32),
        pltpu.VMEM((K,), jnp.int32),
        pltpu.VMEM((K,), jnp.int32),
        pltpu.VMEM((K,), jnp.int32),
        pltpu.VMEM((K,), jnp.int32),
        pltpu.VMEM((K,), jnp.int32),
        pltpu.VMEM((K,), jnp.int32),
        pltpu.VMEM((K, D), jnp.float32),
        pltpu.VMEM((K, D), jnp.float32),
        pltpu.VMEM((K, D), jnp.float32),
        pltpu.VMEM_SHARED((N, D), jnp.float32),
        pltpu.SemaphoreType.DMA,
        pltpu.SemaphoreType.DMA,
        pltpu.SemaphoreType.DMA,
        pltpu.SemaphoreType.DMA,
        pltpu.SemaphoreType.DMA,
        pltpu.SemaphoreType.DMA,
    ],
)
def _agg_kernel(src_hbm, dst_hbm, gm, gv, acc_out, src_seg, dst_seg,
                sidx0, sidx1, sidx2, didx0, didx1, didx2,
                rows0, rows1, rows2,
                acc_sh, gsem0, gsem1, gsem2, ssem0, ssem1, ssem2):
  c = lax.axis_index("c")
  s = lax.axis_index("s")
  sidx = (sidx0, sidx1, sidx2)
  didx = (didx0, didx1, didx2)
  rows = (rows0, rows1, rows2)
  gsems = (gsem0, gsem1, gsem2)
  ssems = (ssem0, ssem1, ssem2)

  rsl = pl.ds(s * ROWS_PER_TILE, ROWS_PER_TILE)
  lsl = pl.ds((NS - 1) * ROWS_PER_TILE, LAST_ROWS)

  @pl.when(jnp.logical_and(c == 0, s < NS - 1))
  def _():
    pltpu.sync_copy(gm.at[rsl], acc_sh.at[rsl])

  @pl.when(jnp.logical_and(c == 0, s == NS - 1))
  def _():
    pltpu.sync_copy(gm.at[lsl], acc_sh.at[lsl])

  @pl.when(jnp.logical_and(c == 1, s < NS - 1))
  def _():
    pltpu.sync_copy(gv.at[rsl], acc_sh.at[rsl])

  @pl.when(jnp.logical_and(c == 1, s == NS - 1))
  def _():
    pltpu.sync_copy(gv.at[lsl], acc_sh.at[lsl])

  plsc.subcore_barrier()

  def refill(seg_i):
    ssl = pl.ds(s * E_PER_TILE + seg_i * SEG_E, SEG_E)
    pltpu.sync_copy(src_hbm.at[ssl], src_seg)
    pltpu.sync_copy(dst_hbm.at[ssl], dst_seg)

  def stage(b, k):
    base = (k % SEG) * K
    _copy_idx_chunk(src_seg, sidx[b], base)
    _copy_idx_chunk(dst_seg, didx[b], base)

  def gather_start(b):
    @pl.when(c == 0)
    def _():
      pltpu.async_copy(gm.at[sidx[b]], rows[b], gsems[b])

    @pl.when(c == 1)
    def _():
      pltpu.async_copy(gv.at[sidx[b]], rows[b], gsems[b])

  def gather_wait(b):
    pltpu.make_async_copy(gm.at[sidx[b]], rows[b], gsems[b]).wait()

  def scatter_start(b):
    pltpu.async_copy(rows[b], acc_sh.at[didx[b]], ssems[b], add=True)

  def scatter_wait(b):
    pltpu.make_async_copy(rows[b], acc_sh.at[didx[b]], ssems[b]).wait()

  refill(0)
  for b in range(2):
    stage(b, b)
    gather_start(b)

  def chunk_body(k, q):
    qn = (q + 2) % 3
    gather_wait(q)

    @pl.when(k >= 1)
    def _():
      scatter_wait(qn)

    scatter_start(q)
    kn = k + 2

    @pl.when(kn < CHUNKS)
    def _():
      @pl.when(kn % SEG == 0)
      def _():
        refill(kn // SEG)

      stage(qn, kn)
      gather_start(qn)

  def outer(o, carry):
    for b in range(3):
      chunk_body(o * 3 + b, b)
    return carry

  lax.fori_loop(0, CHUNKS // 3, outer, 0)
  chunk_body(CHUNKS - 1, 0)
  scatter_wait(0)
  plsc.subcore_barrier()

  @pl.when(s < NS - 1)
  def _():
    pltpu.sync_copy(acc_sh.at[rsl], acc_out.at[c, rsl])

  @pl.when(s == NS - 1)
  def _():
    pltpu.sync_copy(acc_sh.at[lsl], acc_out.at[c, lsl])


def _final_body(acc_ref, deg_ref, om_ref, ov_ref):
  b = lax.rsqrt(deg_ref[...][:, 1:2] + 1.0)
  om_ref[...] = b * acc_ref[0]
  ov_ref[...] = (b * b) * acc_ref[1]


_final_call = pl.pallas_call(
    _final_body,
    grid=(GRID_M,),
    in_specs=[
        pl.BlockSpec((2, BLK, D), lambda i: (0, i, 0)),
        pl.BlockSpec((BLK, 2), lambda i: (i, 0)),
    ],
    out_specs=[
        pl.BlockSpec((BLK, D), lambda i: (i, 0)),
        pl.BlockSpec((BLK, D), lambda i: (i, 0)),
    ],
    out_shape=[
        jax.ShapeDtypeStruct((N, D), jnp.float32),
        jax.ShapeDtypeStruct((N, D), jnp.float32),
    ],
)


def kernel(feat, edge_index, weight_mean, weight_var):
  src = edge_index[0]
  dst = edge_index[1]
  deg = _deg_kernel(edge_index.reshape(-1))
  degcat = jnp.stack([deg[:N], deg[PAD_N:PAD_N + N]], axis=1)
  g_mean, g_var = _dense_call(feat, weight_mean, weight_var, degcat)
  acc = _agg_kernel(src, dst, g_mean, g_var)
  out_mean, out_var = _final_call(acc, degcat)
  return (out_mean, out_var)


# speedup vs baseline: 1.1216x; 1.0046x over previous
import functools

import jax
import jax.numpy as jnp
from jax import lax
from jax.experimental import pallas as pl
from jax.experimental.pallas import tpu as pltpu
from jax.experimental.pallas import tpu_sc as plsc

N = 10000
E = 320000
D = 128
GAMMA = 1.0

NC = 2
NS = 16

PAD_N = 10240
HIST_SLICE = PAD_N // NS
ROWS_PER_TILE = PAD_N // NS
LAST_ROWS = N - (NS - 1) * ROWS_PER_TILE

E_PER_TILE = E // NS
K = 80
CHUNKS = E_PER_TILE // K
SEG = 50
SEG_E = SEG * K

_mesh = plsc.VectorSubcoreMesh(
    core_axis_name="c", subcore_axis_name="s", num_cores=NC, num_subcores=NS)


def _copy_idx_chunk(big, small, base):
  for i in range(K // 16):
    small[pl.ds(i * 16, 16)] = big[pl.ds(base + i * 16, 16)]


@functools.partial(
    pl.kernel,
    out_type=jax.ShapeDtypeStruct((2 * PAD_N,), jnp.float32),
    mesh=_mesh,
    scratch_types=[
        pltpu.VMEM((E_PER_TILE,), jnp.int32),
        pltpu.VMEM((K,), jnp.int32),
        pltpu.VMEM((K,), jnp.int32),
        pltpu.VMEM((K,), jnp.float32),
        pltpu.VMEM((HIST_SLICE,), jnp.float32),
        pltpu.VMEM_SHARED((PAD_N,), jnp.float32),
        pltpu.SemaphoreType.DMA,
        pltpu.SemaphoreType.DMA,
    ],
)
def _deg_kernel(edges_flat, deg_out, idx_big, idx0, idx1, ones_v, zeros_v,
                hist_sh, sem0, sem1):
  c = lax.axis_index("c")
  s = lax.axis_index("s")
  idx_small = (idx0, idx1)
  sems = (sem0, sem1)

  for i in range(K // 16):
    ones_v[pl.ds(i * 16, 16)] = jnp.full((16,), 1.0, jnp.float32)
  for i in range(HIST_SLICE // 16):
    zeros_v[pl.ds(i * 16, 16)] = jnp.full((16,), 0.0, jnp.float32)
  pltpu.sync_copy(zeros_v, hist_sh.at[pl.ds(s * HIST_SLICE, HIST_SLICE)])
  pltpu.sync_copy(edges_flat.at[pl.ds(c * E + s * E_PER_TILE, E_PER_TILE)],
                  idx_big)
  plsc.subcore_barrier()

  def fire(b):
    pltpu.async_copy(ones_v, hist_sh.at[idx_small[b]], sems[b], add=True)

  def drain(b):
    pltpu.make_async_copy(ones_v, hist_sh.at[idx_small[b]], sems[b]).wait()

  for b in range(2):
    _copy_idx_chunk(idx_big, idx_small[b], b * K)
    fire(b)

  def outer(o, carry):
    for b in range(2):
      k = o * 2 + b
      drain(b)
      kn = k + 2

      @pl.when(kn < CHUNKS)
      def _():
        _copy_idx_chunk(idx_big, idx_small[b], kn * K)
        fire(b)

    return carry

  lax.fori_loop(0, CHUNKS // 2, outer, 0)
  plsc.subcore_barrier()
  pltpu.sync_copy(hist_sh.at[pl.ds(s * HIST_SLICE, HIST_SLICE)],
                  deg_out.at[pl.ds(c * PAD_N + s * HIST_SLICE, HIST_SLICE)])


BLK = 2000
GRID_M = N // BLK


def _dense_body(feat_ref, wm_ref, wv_ref, deg_ref, gm_ref, gv_ref):
  x = feat_ref[...]
  mean = jnp.maximum(jnp.dot(x, wm_ref[...],
                             preferred_element_type=jnp.float32), 0.0)
  var = jnp.maximum(jnp.dot(x, wv_ref[...],
                            preferred_element_type=jnp.float32), 0.0)
  att = jnp.exp(-GAMMA * var)
  a = lax.rsqrt(deg_ref[...][:, 0:1] + 1.0)
  gm_ref[...] = a * (mean * att)
  gv_ref[...] = (a * a) * (var * att * att)


_dense_call = pl.pallas_call(
    _dense_body,
    grid=(GRID_M,),
    in_specs=[
        pl.BlockSpec((BLK, D), lambda i: (i, 0)),
        pl.BlockSpec((D, D), lambda i: (0, 0)),
        pl.BlockSpec((D, D), lambda i: (0, 0)),
        pl.BlockSpec((BLK, 2), lambda i: (i, 0)),
    ],
    out_specs=[
        pl.BlockSpec((BLK, D), lambda i: (i, 0)),
        pl.BlockSpec((BLK, D), lambda i: (i, 0)),
    ],
    out_shape=[
        jax.ShapeDtypeStruct((N, D), jnp.float32),
        jax.ShapeDtypeStruct((N, D), jnp.float32),
    ],
)


@functools.partial(
    pl.kernel,
    out_type=jax.ShapeDtypeStruct((2, PAD_N, D), jnp.float32),
    mesh=_mesh,
    scratch_types=[
        pltpu.VMEM((SEG_E,), jnp.int32),
        pltpu.VMEM((SEG_E,), jnp.int32),
        pltpu.VMEM((K,), jnp.int32),
        pltpu.VMEM((K,), jnp.int32),
        pltpu.VMEM((K,), jnp.int32),
        pltpu.VMEM((K,), jnp.int32),
        pltpu.VMEM((K,), jnp.int32),
        pltpu.VMEM((K,), jnp.int32),
        pltpu.VMEM((K, D), jnp.float32),
        pltpu.VMEM((K, D), jnp.float32),
        pltpu.VMEM((K, D), jnp.float32),
        pltpu.VMEM_SHARED((N, D), jnp.float32),
        pltpu.SemaphoreType.DMA,
        pltpu.SemaphoreType.DMA,
        pltpu.SemaphoreType.DMA,
        pltpu.SemaphoreType.DMA,
        pltpu.SemaphoreType.DMA,
        pltpu.SemaphoreType.DMA,
    ],
)
def _agg_kernel(src_hbm, dst_hbm, gm, gv, acc_out, src_seg, dst_seg,
                sidx0, sidx1, sidx2, didx0, didx1, didx2,
                rows0, rows1, rows2,
                acc_sh, gsem0, gsem1, gsem2, ssem0, ssem1, ssem2):
  c = lax.axis_index("c")
  s = lax.axis_index("s")
  sidx = (sidx0, sidx1, sidx2)
  didx = (didx0, didx1, didx2)
  rows = (rows0, rows1, rows2)
  gsems = (gsem0, gsem1, gsem2)
  ssems = (ssem0, ssem1, ssem2)

  rsl = pl.ds(s * ROWS_PER_TILE, ROWS_PER_TILE)
  lsl = pl.ds((NS - 1) * ROWS_PER_TILE, LAST_ROWS)

  def init_acc():
    @pl.when(jnp.logical_and(c == 0, s < NS - 1))
    def _():
      pltpu.sync_copy(gm.at[rsl], acc_sh.at[rsl])

    @pl.when(jnp.logical_and(c == 0, s == NS - 1))
    def _():
      pltpu.sync_copy(gm.at[lsl], acc_sh.at[lsl])

    @pl.when(jnp.logical_and(c == 1, s < NS - 1))
    def _():
      pltpu.sync_copy(gv.at[rsl], acc_sh.at[rsl])

    @pl.when(jnp.logical_and(c == 1, s == NS - 1))
    def _():
      pltpu.sync_copy(gv.at[lsl], acc_sh.at[lsl])

  def refill(seg_i):
    ssl = pl.ds(s * E_PER_TILE + seg_i * SEG_E, SEG_E)
    pltpu.sync_copy(src_hbm.at[ssl], src_seg)
    pltpu.sync_copy(dst_hbm.at[ssl], dst_seg)

  def stage(b, k):
    base = (k % SEG) * K
    _copy_idx_chunk(src_seg, sidx[b], base)
    _copy_idx_chunk(dst_seg, didx[b], base)

  def gather_start(b):
    @pl.when(c == 0)
    def _():
      pltpu.async_copy(gm.at[sidx[b]], rows[b], gsems[b])

    @pl.when(c == 1)
    def _():
      pltpu.async_copy(gv.at[sidx[b]], rows[b], gsems[b])

  def gather_wait(b):
    pltpu.make_async_copy(gm.at[sidx[b]], rows[b], gsems[b]).wait()

  def scatter_start(b):
    pltpu.async_copy(rows[b], acc_sh.at[didx[b]], ssems[b], add=True)

  def scatter_wait(b):
    pltpu.make_async_copy(rows[b], acc_sh.at[didx[b]], ssems[b]).wait()

  refill(0)
  for b in range(2):
    stage(b, b)
    gather_start(b)
  init_acc()
  plsc.subcore_barrier()

  def chunk_body(k, q):
    qn = (q + 2) % 3
    gather_wait(q)

    @pl.when(k >= 1)
    def _():
      scatter_wait(qn)

    scatter_start(q)
    kn = k + 2

    @pl.when(kn < CHUNKS)
    def _():
      @pl.when(kn % SEG == 0)
      def _():
        refill(kn // SEG)

      stage(qn, kn)
      gather_start(qn)

  def outer(o, carry):
    for b in range(3):
      chunk_body(o * 3 + b, b)
    return carry

  lax.fori_loop(0, CHUNKS // 3, outer, 0)
  chunk_body(CHUNKS - 1, 0)
  scatter_wait(0)
  plsc.subcore_barrier()

  @pl.when(s < NS - 1)
  def _():
    pltpu.sync_copy(acc_sh.at[rsl], acc_out.at[c, rsl])

  @pl.when(s == NS - 1)
  def _():
    pltpu.sync_copy(acc_sh.at[lsl], acc_out.at[c, lsl])


def _final_body(acc_ref, deg_ref, om_ref, ov_ref):
  b = lax.rsqrt(deg_ref[...][:, 1:2] + 1.0)
  om_ref[...] = b * acc_ref[0]
  ov_ref[...] = (b * b) * acc_ref[1]


_final_call = pl.pallas_call(
    _final_body,
    grid=(GRID_M,),
    in_specs=[
        pl.BlockSpec((2, BLK, D), lambda i: (0, i, 0)),
        pl.BlockSpec((BLK, 2), lambda i: (i, 0)),
    ],
    out_specs=[
        pl.BlockSpec((BLK, D), lambda i: (i, 0)),
        pl.BlockSpec((BLK, D), lambda i: (i, 0)),
    ],
    out_shape=[
        jax.ShapeDtypeStruct((N, D), jnp.float32),
        jax.ShapeDtypeStruct((N, D), jnp.float32),
    ],
)


def kernel(feat, edge_index, weight_mean, weight_var):
  src = edge_index[0]
  dst = edge_index[1]
  deg = _deg_kernel(edge_index.reshape(-1))
  degcat = jnp.stack([deg[:N], deg[PAD_N:PAD_N + N]], axis=1)
  g_mean, g_var = _dense_call(feat, weight_mean, weight_var, degcat)
  acc = _agg_kernel(src, dst, g_mean, g_var)
  out_mean, out_var = _final_call(acc, degcat)
  return (out_mean, out_var)


# speedup vs baseline: 1.1358x; 1.0127x over previous
import functools

import jax
import jax.numpy as jnp
from jax import lax
from jax.experimental import pallas as pl
from jax.experimental.pallas import tpu as pltpu
from jax.experimental.pallas import tpu_sc as plsc

N = 10000
E = 320000
D = 128
GAMMA = 1.0

NC = 2
NS = 16

PAD_N = 10240
HIST_SLICE = PAD_N // NS
ROWS_PER_TILE = PAD_N // NS
LAST_ROWS = N - (NS - 1) * ROWS_PER_TILE

E_PER_TILE = E // NS
K = 80
CHUNKS = E_PER_TILE // K
SEG = 50
SEG_E = SEG * K
HK = 128
H_FULL = E_PER_TILE // HK
H_TAIL = E_PER_TILE - H_FULL * HK

_mesh = plsc.VectorSubcoreMesh(
    core_axis_name="c", subcore_axis_name="s", num_cores=NC, num_subcores=NS)


def _copy_idx_chunk(big, small, base):
  for i in range(K // 16):
    small[pl.ds(i * 16, 16)] = big[pl.ds(base + i * 16, 16)]


@functools.partial(
    pl.kernel,
    out_type=jax.ShapeDtypeStruct((2 * PAD_N,), jnp.float32),
    mesh=_mesh,
    scratch_types=[
        pltpu.VMEM((E_PER_TILE,), jnp.int32),
        pltpu.VMEM((HK,), jnp.int32),
        pltpu.VMEM((HK,), jnp.int32),
        pltpu.VMEM((H_TAIL,), jnp.int32),
        pltpu.VMEM((HK,), jnp.float32),
        pltpu.VMEM((HIST_SLICE,), jnp.float32),
        pltpu.VMEM_SHARED((PAD_N,), jnp.float32),
        pltpu.SemaphoreType.DMA,
        pltpu.SemaphoreType.DMA,
    ],
)
def _deg_kernel(edges_flat, deg_out, idx_big, idx0, idx1, idx_tail,
                ones_v, zeros_v, hist_sh, sem0, sem1):
  c = lax.axis_index("c")
  s = lax.axis_index("s")
  idx_small = (idx0, idx1)
  sems = (sem0, sem1)

  for i in range(HK // 16):
    ones_v[pl.ds(i * 16, 16)] = jnp.full((16,), 1.0, jnp.float32)
  for i in range(HIST_SLICE // 16):
    zeros_v[pl.ds(i * 16, 16)] = jnp.full((16,), 0.0, jnp.float32)
  pltpu.sync_copy(zeros_v, hist_sh.at[pl.ds(s * HIST_SLICE, HIST_SLICE)])
  pltpu.sync_copy(edges_flat.at[pl.ds(c * E + s * E_PER_TILE, E_PER_TILE)],
                  idx_big)
  plsc.subcore_barrier()

  def copy_chunk(b, base):
    for i in range(HK // 16):
      idx_small[b][pl.ds(i * 16, 16)] = idx_big[pl.ds(base + i * 16, 16)]

  def fire(b):
    pltpu.async_copy(ones_v, hist_sh.at[idx_small[b]], sems[b], add=True)

  def drain(b):
    pltpu.make_async_copy(ones_v, hist_sh.at[idx_small[b]], sems[b]).wait()

  for b in range(2):
    copy_chunk(b, b * HK)
    fire(b)

  def outer(o, carry):
    for b in range(2):
      k = o * 2 + b
      drain(b)
      kn = k + 2

      @pl.when(kn < H_FULL)
      def _():
        copy_chunk(b, kn * HK)
        fire(b)

    return carry

  lax.fori_loop(0, H_FULL // 2, outer, 0)
  for i in range(H_TAIL // 16):
    idx_tail[pl.ds(i * 16, 16)] = idx_big[pl.ds(H_FULL * HK + i * 16, 16)]
  pltpu.sync_copy(ones_v.at[pl.ds(0, H_TAIL)], hist_sh.at[idx_tail], add=True)
  plsc.subcore_barrier()
  pltpu.sync_copy(hist_sh.at[pl.ds(s * HIST_SLICE, HIST_SLICE)],
                  deg_out.at[pl.ds(c * PAD_N + s * HIST_SLICE, HIST_SLICE)])


BLK = 2000
GRID_M = N // BLK


def _dense_body(feat_ref, wm_ref, wv_ref, deg_ref, gm_ref, gv_ref):
  x = feat_ref[...]
  mean = jnp.maximum(jnp.dot(x, wm_ref[...],
                             preferred_element_type=jnp.float32), 0.0)
  var = jnp.maximum(jnp.dot(x, wv_ref[...],
                            preferred_element_type=jnp.float32), 0.0)
  att = jnp.exp(-GAMMA * var)
  a = lax.rsqrt(deg_ref[...][:, 0:1] + 1.0)
  gm_ref[...] = a * (mean * att)
  gv_ref[...] = (a * a) * (var * att * att)


_dense_call = pl.pallas_call(
    _dense_body,
    grid=(GRID_M,),
    in_specs=[
        pl.BlockSpec((BLK, D), lambda i: (i, 0)),
        pl.BlockSpec((D, D), lambda i: (0, 0)),
        pl.BlockSpec((D, D), lambda i: (0, 0)),
        pl.BlockSpec((BLK, 2), lambda i: (i, 0)),
    ],
    out_specs=[
        pl.BlockSpec((BLK, D), lambda i: (i, 0)),
        pl.BlockSpec((BLK, D), lambda i: (i, 0)),
    ],
    out_shape=[
        jax.ShapeDtypeStruct((N, D), jnp.float32),
        jax.ShapeDtypeStruct((N, D), jnp.float32),
    ],
)


@functools.partial(
    pl.kernel,
    out_type=jax.ShapeDtypeStruct((2, PAD_N, D), jnp.float32),
    mesh=_mesh,
    scratch_types=[
        pltpu.VMEM((SEG_E,), jnp.int32),
        pltpu.VMEM((SEG_E,), jnp.int32),
        pltpu.VMEM((K,), jnp.int32),
        pltpu.VMEM((K,), jnp.int32),
        pltpu.VMEM((K,), jnp.int32),
        pltpu.VMEM((K,), jnp.int32),
        pltpu.VMEM((K,), jnp.int32),
        pltpu.VMEM((K,), jnp.int32),
        pltpu.VMEM((K, D), jnp.float32),
        pltpu.VMEM((K, D), jnp.float32),
        pltpu.VMEM((K, D), jnp.float32),
        pltpu.VMEM_SHARED((N, D), jnp.float32),
        pltpu.SemaphoreType.DMA,
        pltpu.SemaphoreType.DMA,
        pltpu.SemaphoreType.DMA,
        pltpu.SemaphoreType.DMA,
        pltpu.SemaphoreType.DMA,
        pltpu.SemaphoreType.DMA,
    ],
)
def _agg_kernel(src_hbm, dst_hbm, gm, gv, acc_out, src_seg, dst_seg,
                sidx0, sidx1, sidx2, didx0, didx1, didx2,
                rows0, rows1, rows2,
                acc_sh, gsem0, gsem1, gsem2, ssem0, ssem1, ssem2):
  c = lax.axis_index("c")
  s = lax.axis_index("s")
  sidx = (sidx0, sidx1, sidx2)
  didx = (didx0, didx1, didx2)
  rows = (rows0, rows1, rows2)
  gsems = (gsem0, gsem1, gsem2)
  ssems = (ssem0, ssem1, ssem2)

  rsl = pl.ds(s * ROWS_PER_TILE, ROWS_PER_TILE)
  lsl = pl.ds((NS - 1) * ROWS_PER_TILE, LAST_ROWS)

  def init_acc():
    @pl.when(jnp.logical_and(c == 0, s < NS - 1))
    def _():
      pltpu.sync_copy(gm.at[rsl], acc_sh.at[rsl])

    @pl.when(jnp.logical_and(c == 0, s == NS - 1))
    def _():
      pltpu.sync_copy(gm.at[lsl], acc_sh.at[lsl])

    @pl.when(jnp.logical_and(c == 1, s < NS - 1))
    def _():
      pltpu.sync_copy(gv.at[rsl], acc_sh.at[rsl])

    @pl.when(jnp.logical_and(c == 1, s == NS - 1))
    def _():
      pltpu.sync_copy(gv.at[lsl], acc_sh.at[lsl])

  def refill(seg_i):
    ssl = pl.ds(s * E_PER_TILE + seg_i * SEG_E, SEG_E)
    pltpu.sync_copy(src_hbm.at[ssl], src_seg)
    pltpu.sync_copy(dst_hbm.at[ssl], dst_seg)

  def stage(b, k):
    base = (k % SEG) * K
    _copy_idx_chunk(src_seg, sidx[b], base)
    _copy_idx_chunk(dst_seg, didx[b], base)

  def gather_start(b):
    @pl.when(c == 0)
    def _():
      pltpu.async_copy(gm.at[sidx[b]], rows[b], gsems[b])

    @pl.when(c == 1)
    def _():
      pltpu.async_copy(gv.at[sidx[b]], rows[b], gsems[b])

  def gather_wait(b):
    pltpu.make_async_copy(gm.at[sidx[b]], rows[b], gsems[b]).wait()

  def scatter_start(b):
    pltpu.async_copy(rows[b], acc_sh.at[didx[b]], ssems[b], add=True)

  def scatter_wait(b):
    pltpu.make_async_copy(rows[b], acc_sh.at[didx[b]], ssems[b]).wait()

  refill(0)
  for b in range(2):
    stage(b, b)
    gather_start(b)
  init_acc()
  plsc.subcore_barrier()

  def chunk_body(k, q):
    qn = (q + 2) % 3
    gather_wait(q)

    @pl.when(k >= 1)
    def _():
      scatter_wait(qn)

    scatter_start(q)
    kn = k + 2

    @pl.when(kn < CHUNKS)
    def _():
      @pl.when(kn % SEG == 0)
      def _():
        refill(kn // SEG)

      stage(qn, kn)
      gather_start(qn)

  def outer(o, carry):
    for b in range(3):
      chunk_body(o * 3 + b, b)
    return carry

  lax.fori_loop(0, CHUNKS // 3, outer, 0)
  chunk_body(CHUNKS - 1, 0)
  scatter_wait(0)
  plsc.subcore_barrier()

  @pl.when(s < NS - 1)
  def _():
    pltpu.sync_copy(acc_sh.at[rsl], acc_out.at[c, rsl])

  @pl.when(s == NS - 1)
  def _():
    pltpu.sync_copy(acc_sh.at[lsl], acc_out.at[c, lsl])


def _final_body(acc_ref, deg_ref, om_ref, ov_ref):
  b = lax.rsqrt(deg_ref[...][:, 1:2] + 1.0)
  om_ref[...] = b * acc_ref[0]
  ov_ref[...] = (b * b) * acc_ref[1]


_final_call = pl.pallas_call(
    _final_body,
    grid=(GRID_M,),
    in_specs=[
        pl.BlockSpec((2, BLK, D), lambda i: (0, i, 0)),
        pl.BlockSpec((BLK, 2), lambda i: (i, 0)),
    ],
    out_specs=[
        pl.BlockSpec((BLK, D), lambda i: (i, 0)),
        pl.BlockSpec((BLK, D), lambda i: (i, 0)),
    ],
    out_shape=[
        jax.ShapeDtypeStruct((N, D), jnp.float32),
        jax.ShapeDtypeStruct((N, D), jnp.float32),
    ],
)


def kernel(feat, edge_index, weight_mean, weight_var):
  src = edge_index[0]
  dst = edge_index[1]
  deg = _deg_kernel(edge_index.reshape(-1))
  degcat = jnp.stack([deg[:N], deg[PAD_N:PAD_N + N]], axis=1)
  g_mean, g_var = _dense_call(feat, weight_mean, weight_var, degcat)
  acc = _agg_kernel(src, dst, g_mean, g_var)
  out_mean, out_var = _final_call(acc, degcat)
  return (out_mean, out_var)


# speedup vs baseline: 1.1718x; 1.0316x over previous
import functools

import jax
import jax.numpy as jnp
from jax import lax
from jax.experimental import pallas as pl
from jax.experimental.pallas import tpu as pltpu
from jax.experimental.pallas import tpu_sc as plsc

N = 10000
E = 320000
D = 128
GAMMA = 1.0

NC = 2
NS = 16

PAD_N = 10240
HIST_SLICE = PAD_N // NS
ROWS_PER_TILE = PAD_N // NS
LAST_ROWS = N - (NS - 1) * ROWS_PER_TILE

E_PER_TILE = E // NS
K = 80
CHUNKS = E_PER_TILE // K
SEG = 50
SEG_E = SEG * K
HK = 128
H_FULL = E_PER_TILE // HK
H_TAIL = E_PER_TILE - H_FULL * HK

_mesh = plsc.VectorSubcoreMesh(
    core_axis_name="c", subcore_axis_name="s", num_cores=NC, num_subcores=NS)


def _copy_idx_chunk(big, small, base):
  for i in range(K // 16):
    small[pl.ds(i * 16, 16)] = big[pl.ds(base + i * 16, 16)]


@functools.partial(
    pl.kernel,
    out_type=jax.ShapeDtypeStruct((2 * PAD_N,), jnp.float32),
    mesh=_mesh,
    scratch_types=[
        pltpu.VMEM((E_PER_TILE,), jnp.int32),
        pltpu.VMEM((HK,), jnp.int32),
        pltpu.VMEM((HK,), jnp.int32),
        pltpu.VMEM((H_TAIL,), jnp.int32),
        pltpu.VMEM((HK,), jnp.float32),
        pltpu.VMEM((HIST_SLICE,), jnp.float32),
        pltpu.VMEM_SHARED((PAD_N,), jnp.float32),
        pltpu.SemaphoreType.DMA,
        pltpu.SemaphoreType.DMA,
    ],
)
def _deg_kernel(edges_flat, deg_out, idx_big, idx0, idx1, idx_tail,
                ones_v, zeros_v, hist_sh, sem0, sem1):
  c = lax.axis_index("c")
  s = lax.axis_index("s")
  idx_small = (idx0, idx1)
  sems = (sem0, sem1)

  for i in range(HK // 16):
    ones_v[pl.ds(i * 16, 16)] = jnp.full((16,), 1.0, jnp.float32)
  for i in range(HIST_SLICE // 16):
    zeros_v[pl.ds(i * 16, 16)] = jnp.full((16,), 0.0, jnp.float32)
  pltpu.sync_copy(zeros_v, hist_sh.at[pl.ds(s * HIST_SLICE, HIST_SLICE)])
  pltpu.sync_copy(edges_flat.at[pl.ds(c * E + s * E_PER_TILE, E_PER_TILE)],
                  idx_big)
  plsc.subcore_barrier()

  def copy_chunk(b, base):
    for i in range(HK // 16):
      idx_small[b][pl.ds(i * 16, 16)] = idx_big[pl.ds(base + i * 16, 16)]

  def fire(b):
    pltpu.async_copy(ones_v, hist_sh.at[idx_small[b]], sems[b], add=True)

  def drain(b):
    pltpu.make_async_copy(ones_v, hist_sh.at[idx_small[b]], sems[b]).wait()

  for b in range(2):
    copy_chunk(b, b * HK)
    fire(b)

  def outer(o, carry):
    for b in range(2):
      k = o * 2 + b
      drain(b)
      kn = k + 2

      @pl.when(kn < H_FULL)
      def _():
        copy_chunk(b, kn * HK)
        fire(b)

    return carry

  lax.fori_loop(0, H_FULL // 2, outer, 0)
  for i in range(H_TAIL // 16):
    idx_tail[pl.ds(i * 16, 16)] = idx_big[pl.ds(H_FULL * HK + i * 16, 16)]
  pltpu.sync_copy(ones_v.at[pl.ds(0, H_TAIL)], hist_sh.at[idx_tail], add=True)
  plsc.subcore_barrier()
  pltpu.sync_copy(hist_sh.at[pl.ds(s * HIST_SLICE, HIST_SLICE)],
                  deg_out.at[pl.ds(c * PAD_N + s * HIST_SLICE, HIST_SLICE)])


BLK = 2048
GRID_M = PAD_N // BLK
DEG_R = BLK // D


def _dense_body(feat_ref, wm_ref, wv_ref, deg_ref, gm_ref, gv_ref):
  x = feat_ref[...]
  mean = jnp.maximum(jnp.dot(x, wm_ref[...],
                             preferred_element_type=jnp.float32), 0.0)
  var = jnp.maximum(jnp.dot(x, wv_ref[...],
                            preferred_element_type=jnp.float32), 0.0)
  att = jnp.exp(-GAMMA * var)
  a = lax.rsqrt(deg_ref[0] + 1.0)
  a3 = lax.broadcast_in_dim(a, (DEG_R, D, D), (0, 1))
  gm3 = a3 * (mean * att).reshape(DEG_R, D, D)
  gv3 = (a3 * a3) * (var * att * att).reshape(DEG_R, D, D)
  gm_ref[...] = gm3.reshape(BLK, D)
  gv_ref[...] = gv3.reshape(BLK, D)


_dense_call = pl.pallas_call(
    _dense_body,
    grid=(GRID_M,),
    in_specs=[
        pl.BlockSpec((BLK, D), lambda i: (i, 0)),
        pl.BlockSpec((D, D), lambda i: (0, 0)),
        pl.BlockSpec((D, D), lambda i: (0, 0)),
        pl.BlockSpec((1, DEG_R, D), lambda i: (0, i, 0)),
    ],
    out_specs=[
        pl.BlockSpec((BLK, D), lambda i: (i, 0)),
        pl.BlockSpec((BLK, D), lambda i: (i, 0)),
    ],
    out_shape=[
        jax.ShapeDtypeStruct((N, D), jnp.float32),
        jax.ShapeDtypeStruct((N, D), jnp.float32),
    ],
)


@functools.partial(
    pl.kernel,
    out_type=jax.ShapeDtypeStruct((2, PAD_N, D), jnp.float32),
    mesh=_mesh,
    scratch_types=[
        pltpu.VMEM((SEG_E,), jnp.int32),
        pltpu.VMEM((SEG_E,), jnp.int32),
        pltpu.VMEM((K,), jnp.int32),
        pltpu.VMEM((K,), jnp.int32),
        pltpu.VMEM((K,), jnp.int32),
        pltpu.VMEM((K,), jnp.int32),
        pltpu.VMEM((K,), jnp.int32),
        pltpu.VMEM((K,), jnp.int32),
        pltpu.VMEM((K, D), jnp.float32),
        pltpu.VMEM((K, D), jnp.float32),
        pltpu.VMEM((K, D), jnp.float32),
        pltpu.VMEM_SHARED((N, D), jnp.float32),
        pltpu.SemaphoreType.DMA,
        pltpu.SemaphoreType.DMA,
        pltpu.SemaphoreType.DMA,
        pltpu.SemaphoreType.DMA,
        pltpu.SemaphoreType.DMA,
        pltpu.SemaphoreType.DMA,
    ],
)
def _agg_kernel(src_hbm, dst_hbm, gm, gv, acc_out, src_seg, dst_seg,
                sidx0, sidx1, sidx2, didx0, didx1, didx2,
                rows0, rows1, rows2,
                acc_sh, gsem0, gsem1, gsem2, ssem0, ssem1, ssem2):
  c = lax.axis_index("c")
  s = lax.axis_index("s")
  sidx = (sidx0, sidx1, sidx2)
  didx = (didx0, didx1, didx2)
  rows = (rows0, rows1, rows2)
  gsems = (gsem0, gsem1, gsem2)
  ssems = (ssem0, ssem1, ssem2)

  rsl = pl.ds(s * ROWS_PER_TILE, ROWS_PER_TILE)
  lsl = pl.ds((NS - 1) * ROWS_PER_TILE, LAST_ROWS)

  def init_acc():
    @pl.when(jnp.logical_and(c == 0, s < NS - 1))
    def _():
      pltpu.sync_copy(gm.at[rsl], acc_sh.at[rsl])

    @pl.when(jnp.logical_and(c == 0, s == NS - 1))
    def _():
      pltpu.sync_copy(gm.at[lsl], acc_sh.at[lsl])

    @pl.when(jnp.logical_and(c == 1, s < NS - 1))
    def _():
      pltpu.sync_copy(gv.at[rsl], acc_sh.at[rsl])

    @pl.when(jnp.logical_and(c == 1, s == NS - 1))
    def _():
      pltpu.sync_copy(gv.at[lsl], acc_sh.at[lsl])

  def refill(seg_i):
    ssl = pl.ds(s * E_PER_TILE + seg_i * SEG_E, SEG_E)
    pltpu.sync_copy(src_hbm.at[ssl], src_seg)
    pltpu.sync_copy(dst_hbm.at[ssl], dst_seg)

  def stage(b, k):
    base = (k % SEG) * K
    _copy_idx_chunk(src_seg, sidx[b], base)
    _copy_idx_chunk(dst_seg, didx[b], base)

  def gather_start(b):
    @pl.when(c == 0)
    def _():
      pltpu.async_copy(gm.at[sidx[b]], rows[b], gsems[b])

    @pl.when(c == 1)
    def _():
      pltpu.async_copy(gv.at[sidx[b]], rows[b], gsems[b])

  def gather_wait(b):
    pltpu.make_async_copy(gm.at[sidx[b]], rows[b], gsems[b]).wait()

  def scatter_start(b):
    pltpu.async_copy(rows[b], acc_sh.at[didx[b]], ssems[b], add=True)

  def scatter_wait(b):
    pltpu.make_async_copy(rows[b], acc_sh.at[didx[b]], ssems[b]).wait()

  refill(0)
  for b in range(2):
    stage(b, b)
    gather_start(b)
  init_acc()
  plsc.subcore_barrier()

  def chunk_body(k, q):
    qn = (q + 2) % 3
    gather_wait(q)

    @pl.when(k >= 1)
    def _():
      scatter_wait(qn)

    scatter_start(q)
    kn = k + 2

    @pl.when(kn < CHUNKS)
    def _():
      @pl.when(kn % SEG == 0)
      def _():
        refill(kn // SEG)

      stage(qn, kn)
      gather_start(qn)

  def outer(o, carry):
    for b in range(3):
      chunk_body(o * 3 + b, b)
    return carry

  lax.fori_loop(0, CHUNKS // 3, outer, 0)
  chunk_body(CHUNKS - 1, 0)
  scatter_wait(0)
  plsc.subcore_barrier()

  @pl.when(s < NS - 1)
  def _():
    pltpu.sync_copy(acc_sh.at[rsl], acc_out.at[c, rsl])

  @pl.when(s == NS - 1)
  def _():
    pltpu.sync_copy(acc_sh.at[lsl], acc_out.at[c, lsl])


def _final_body(acc_ref, deg_ref, om_ref, ov_ref):
  b = lax.rsqrt(deg_ref[0] + 1.0)
  b3 = lax.broadcast_in_dim(b, (DEG_R, D, D), (0, 1))
  om_ref[...] = (b3 * acc_ref[0].reshape(DEG_R, D, D)).reshape(BLK, D)
  ov_ref[...] = (b3 * b3 * acc_ref[1].reshape(DEG_R, D, D)).reshape(BLK, D)


_final_call = pl.pallas_call(
    _final_body,
    grid=(GRID_M,),
    in_specs=[
        pl.BlockSpec((2, BLK, D), lambda i: (0, i, 0)),
        pl.BlockSpec((1, DEG_R, D), lambda i: (1, i, 0)),
    ],
    out_specs=[
        pl.BlockSpec((BLK, D), lambda i: (i, 0)),
        pl.BlockSpec((BLK, D), lambda i: (i, 0)),
    ],
    out_shape=[
        jax.ShapeDtypeStruct((N, D), jnp.float32),
        jax.ShapeDtypeStruct((N, D), jnp.float32),
    ],
)


def kernel(feat, edge_index, weight_mean, weight_var):
  src = edge_index[0]
  dst = edge_index[1]
  deg = _deg_kernel(edge_index.reshape(-1))
  deg3 = deg.reshape(2, PAD_N // D, D)
  g_mean, g_var = _dense_call(feat, weight_mean, weight_var, deg3)
  acc = _agg_kernel(src, dst, g_mean, g_var)
  out_mean, out_var = _final_call(acc, deg3)
  return (out_mean, out_var)
